# baseline (speedup 1.0000x reference)
import functools

import jax
import jax.numpy as jnp
from jax import lax
from jax.experimental import pallas as pl
from jax.experimental.pallas import tpu as pltpu
from jax.experimental.pallas import tpu_sc as plsc

_NUM_EXPERTS = 64
_SCALE = 20.0
_L = 16
_NBLK = _NUM_EXPERTS // _L


def _silu(x):
    return x / (1.0 + jnp.exp(-x))


def _silu_threshold(x):
    d = _SCALE * x
    return (_silu(d + 0.5 * _SCALE) - _silu(d - 0.5 * _SCALE)) / _SCALE


def _score_block(op16, k):
    e = lax.convert_element_type(lax.iota(jnp.int32, _L), jnp.float32) + float(k * _L)
    diff = op16 - e
    return _silu_threshold(diff + 0.5) * _silu_threshold(-diff + 0.5)


def _body(op_hbm, scores_hbm, idx_hbm, op_v, scores_v, idx_v, val_s, idx_s, sem):
    cid = lax.axis_index("c")
    sid = lax.axis_index("s")

    @pl.when(jnp.logical_and(cid == 0, sid == 0))
    def _():
        pltpu.sync_copy(op_hbm, op_v)
        op16 = op_v[...]
        lanes = lax.iota(jnp.int32, _L)
        best_val = _score_block(op16, 0)
        best_idx = lanes
        scores_v[pl.ds(0, _L)] = best_val
        for k in range(1, _NBLK):
            s = _score_block(op16, k)
            scores_v[pl.ds(k * _L, _L)] = s
            idx_k = lanes + (k * _L)
            upd = s > best_val
            best_val = jnp.where(upd, s, best_val)
            best_idx = jnp.where(upd, idx_k, best_idx)
        scores_dma = pltpu.async_copy(scores_v, scores_hbm, sem)
        for off in (8, 4, 2, 1):
            val_s[...] = best_val
            idx_s[...] = best_idx
            perm = lanes ^ off
            o_val = plsc.load_gather(val_s, [perm])
            o_idx = plsc.load_gather(idx_s, [perm])
            take = jnp.logical_or(
                o_val > best_val,
                jnp.logical_and(o_val == best_val, o_idx < best_idx),
            )
            best_val = jnp.where(take, o_val, best_val)
            best_idx = jnp.where(take, o_idx, best_idx)
        idx_v[...] = best_idx
        pltpu.sync_copy(idx_v, idx_hbm)
        scores_dma.wait()


@functools.cache
def _router():
    return functools.partial(
        pl.kernel,
        out_type=(
            jax.ShapeDtypeStruct((_NUM_EXPERTS,), jnp.float32),
            jax.ShapeDtypeStruct((_L,), jnp.int32),
        ),
        mesh=plsc.VectorSubcoreMesh(
            core_axis_name="c", subcore_axis_name="s", num_cores=1, num_subcores=1
        ),
        compiler_params=pltpu.CompilerParams(needs_layout_passes=False),
        scratch_types=[
            pltpu.VMEM((_L,), jnp.float32),
            pltpu.VMEM((_NUM_EXPERTS,), jnp.float32),
            pltpu.VMEM((_L,), jnp.int32),
            pltpu.VMEM((_L,), jnp.float32),
            pltpu.VMEM((_L,), jnp.int32),
            pltpu.SemaphoreType.DMA,
        ],
    )(_body)


def kernel(opcode):
    op16 = jnp.broadcast_to(opcode.astype(jnp.float32), (_L,))
    scores, idx = _router()(op16)
    return scores, idx[0]


# speedup vs baseline: 1.0025x; 1.0025x over previous
import functools

import jax
import jax.numpy as jnp
from jax import lax
from jax.experimental import pallas as pl
from jax.experimental.pallas import tpu as pltpu
from jax.experimental.pallas import tpu_sc as plsc

_NUM_EXPERTS = 64
_SCALE = 20.0
_L = 16
_NBLK = _NUM_EXPERTS // _L


def _silu(x):
    return x / (1.0 + jnp.exp(-x))


def _silu_threshold(x):
    d = _SCALE * x
    return (_silu(d + 0.5 * _SCALE) - _silu(d - 0.5 * _SCALE)) / _SCALE


def _score_block(op16, k):
    e = lax.convert_element_type(lax.iota(jnp.int32, _L), jnp.float32) + float(k * _L)
    diff = op16 - e
    return _silu_threshold(diff + 0.5) * _silu_threshold(-diff + 0.5)


def _body(op_hbm, scores_hbm, idx_hbm, op_v, scores_v, idx_v, val_s, idx_s, sem):
    pltpu.sync_copy(op_hbm, op_v)
    op16 = op_v[...]
    lanes = lax.iota(jnp.int32, _L)
    best_val = _score_block(op16, 0)
    best_idx = lanes
    scores_v[pl.ds(0, _L)] = best_val
    for k in range(1, _NBLK):
        s = _score_block(op16, k)
        scores_v[pl.ds(k * _L, _L)] = s
        idx_k = lanes + (k * _L)
        upd = s > best_val
        best_val = jnp.where(upd, s, best_val)
        best_idx = jnp.where(upd, idx_k, best_idx)
    scores_dma = pltpu.async_copy(scores_v, scores_hbm, sem)
    for off in (8, 4, 2, 1):
        val_s[...] = best_val
        idx_s[...] = best_idx
        perm = lanes ^ off
        o_val = plsc.load_gather(val_s, [perm])
        o_idx = plsc.load_gather(idx_s, [perm])
        take = jnp.logical_or(
            o_val > best_val,
            jnp.logical_and(o_val == best_val, o_idx < best_idx),
        )
        best_val = jnp.where(take, o_val, best_val)
        best_idx = jnp.where(take, o_idx, best_idx)
    idx_v[...] = best_idx
    pltpu.sync_copy(idx_v, idx_hbm)
    scores_dma.wait()


@functools.cache
def _router():
    return functools.partial(
        pl.kernel,
        out_type=(
            jax.ShapeDtypeStruct((_NUM_EXPERTS,), jnp.float32),
            jax.ShapeDtypeStruct((_L,), jnp.int32),
        ),
        mesh=plsc.VectorSubcoreMesh(
            core_axis_name="c", subcore_axis_name="s", num_cores=1, num_subcores=1
        ),
        compiler_params=pltpu.CompilerParams(needs_layout_passes=False),
        scratch_types=[
            pltpu.VMEM((_L,), jnp.float32),
            pltpu.VMEM((_NUM_EXPERTS,), jnp.float32),
            pltpu.VMEM((_L,), jnp.int32),
            pltpu.VMEM((_L,), jnp.float32),
            pltpu.VMEM((_L,), jnp.int32),
            pltpu.SemaphoreType.DMA,
        ],
    )(_body)


def kernel(opcode):
    op16 = jnp.broadcast_to(opcode.astype(jnp.float32), (_L,))
    scores, idx = _router()(op16)
    return scores, idx[0]
